# Initial kernel scaffold; baseline (speedup 1.0000x reference)
import jax
import jax.numpy as jnp
from jax.experimental import pallas as pl


def kernel(y, table):
    raise NotImplementedError("write your pallas kernel here")


# speedup vs baseline: 1.0164x; 1.0164x over previous
import functools

import jax
import jax.numpy as jnp
from jax import lax
from jax.experimental import pallas as pl
from jax.experimental.pallas import tpu as pltpu
from jax.experimental.pallas import tpu_sc as plsc

NLABELS = 100000
EMBED_DIM = 128
BATCH = 4096

_L = 16
_NW = 32
_BPW = BATCH // _NW
_CHUNKS = EMBED_DIM // _L


_GDN = lax.GatherDimensionNumbers(
    offset_dims=(), collapsed_slice_dims=(0,), start_index_map=(0,)
)


def _permute(v, idx):
    return lax.gather(
        v,
        idx[:, None],
        dimension_numbers=_GDN,
        slice_sizes=(1,),
        mode=lax.GatherScatterMode.PROMISE_IN_BOUNDS,
    )


def _lane_sum(v):
    lanes = lax.iota(jnp.int32, _L)
    for sh in (8, 4, 2, 1):
        v = v + _permute(v, lanes ^ sh)
    return v


def _rsqrt(s):
    i = plsc.bitcast(s, jnp.int32)
    i = jnp.int32(0x5F3759DF) - (i >> 1)
    y = plsc.bitcast(i, jnp.float32)
    half_s = s * 0.5
    for _ in range(3):
        y = y * (1.5 - half_s * y * y)
    return y


def _body(y_hbm, table_hbm, out_hbm, idx_v, rows_v, sem):
    wid = lax.axis_index("s") * 2 + lax.axis_index("c")
    base = wid * _BPW
    pltpu.sync_copy(y_hbm.at[pl.ds(base, _BPW)], idx_v)
    pltpu.async_copy(table_hbm.at[idx_v], rows_v, sem).wait()

    def row(r, _):
        chunks = [rows_v[r, pl.ds(c * _L, _L)] for c in range(_CHUNKS)]
        acc = chunks[0] * chunks[0]
        for c in range(1, _CHUNKS):
            acc = acc + chunks[c] * chunks[c]
        scale = _rsqrt(_lane_sum(acc))
        for c in range(_CHUNKS):
            rows_v[r, pl.ds(c * _L, _L)] = chunks[c] * scale
        return _

    lax.fori_loop(0, _BPW, row, None)
    pltpu.sync_copy(rows_v, out_hbm.at[pl.ds(base, _BPW)])


@jax.jit
def kernel(y, table):
    mesh = plsc.VectorSubcoreMesh(core_axis_name="c", subcore_axis_name="s")
    f = functools.partial(
        pl.kernel,
        mesh=mesh,
        out_type=jax.ShapeDtypeStruct((BATCH, EMBED_DIM), jnp.float32),
        scratch_types=[
            pltpu.VMEM((_BPW,), jnp.int32),
            pltpu.VMEM((_BPW, EMBED_DIM), jnp.float32),
            pltpu.SemaphoreType.DMA,
        ],
        compiler_params=pltpu.CompilerParams(needs_layout_passes=False),
    )(_body)
    return f(y.astype(jnp.int32), table)


# speedup vs baseline: 1.1354x; 1.1170x over previous
import functools

import jax
import jax.numpy as jnp
from jax import lax
from jax.experimental import pallas as pl
from jax.experimental.pallas import tpu as pltpu
from jax.experimental.pallas import tpu_sc as plsc

NLABELS = 100000
EMBED_DIM = 128
BATCH = 4096

_L = 16
_NW = 32
_BPW = BATCH // _NW
_CHUNKS = EMBED_DIM // _L
_NCH = 4
_RPC = _BPW // _NCH
_UNROLL = 4

_GDN = lax.GatherDimensionNumbers(
    offset_dims=(), collapsed_slice_dims=(0,), start_index_map=(0,)
)


def _permute(v, idx):
    return lax.gather(
        v,
        idx[:, None],
        dimension_numbers=_GDN,
        slice_sizes=(1,),
        mode=lax.GatherScatterMode.PROMISE_IN_BOUNDS,
    )


def _lane_sum(v):
    lanes = lax.iota(jnp.int32, _L)
    for sh in (8, 4, 2, 1):
        v = v + _permute(v, lanes ^ sh)
    return v


def _rsqrt(s):
    i = plsc.bitcast(s, jnp.int32)
    i = jnp.int32(0x5F3759DF) - (i >> 1)
    y = plsc.bitcast(i, jnp.float32)
    half_s = s * 0.5
    for _ in range(2):
        y = y * (1.5 - half_s * y * y)
    return y


def _normalize_row(rows_v, r):
    chunks = [rows_v[r, pl.ds(c * _L, _L)] for c in range(_CHUNKS)]
    acc = chunks[0] * chunks[0]
    for c in range(1, _CHUNKS):
        acc = acc + chunks[c] * chunks[c]
    scale = _rsqrt(_lane_sum(acc))
    for c in range(_CHUNKS):
        rows_v[r, pl.ds(c * _L, _L)] = chunks[c] * scale


def _body(y_hbm, table_hbm, out_hbm, idx_v, rows_v, gsems, osems):
    wid = lax.axis_index("s") * 2 + lax.axis_index("c")
    base = wid * _BPW
    pltpu.sync_copy(y_hbm.at[pl.ds(base, _BPW)], idx_v)
    gathers = [
        pltpu.async_copy(
            table_hbm.at[idx_v.at[pl.ds(ch * _RPC, _RPC)]],
            rows_v.at[pl.ds(ch * _RPC, _RPC)],
            gsems.at[ch],
        )
        for ch in range(_NCH)
    ]
    writes = []
    for ch in range(_NCH):
        gathers[ch].wait()

        def group(i, _, ch=ch):
            for k in range(_UNROLL):
                _normalize_row(rows_v, ch * _RPC + i * _UNROLL + k)
            return _

        lax.fori_loop(0, _RPC // _UNROLL, group, None)
        writes.append(
            pltpu.async_copy(
                rows_v.at[pl.ds(ch * _RPC, _RPC)],
                out_hbm.at[pl.ds(base + ch * _RPC, _RPC)],
                osems.at[ch],
            )
        )
    for w in writes:
        w.wait()


@jax.jit
def kernel(y, table):
    mesh = plsc.VectorSubcoreMesh(core_axis_name="c", subcore_axis_name="s")
    f = functools.partial(
        pl.kernel,
        mesh=mesh,
        out_type=jax.ShapeDtypeStruct((BATCH, EMBED_DIM), jnp.float32),
        scratch_types=[
            pltpu.VMEM((_BPW,), jnp.int32),
            pltpu.VMEM((_BPW, EMBED_DIM), jnp.float32),
            pltpu.SemaphoreType.DMA((_NCH,)),
            pltpu.SemaphoreType.DMA((_NCH,)),
        ],
        compiler_params=pltpu.CompilerParams(needs_layout_passes=False),
    )(_body)
    return f(y.astype(jnp.int32), table)
